# baseline (speedup 1.0000x reference)
import jax
import jax.numpy as jnp
from jax import lax
from jax.experimental import pallas as pl
from jax.experimental.pallas import tpu as pltpu


def _moe_fused(idx_ref, ew_ref, act_ref, w_ref, bias_ref, resid_ref, out_ref,
               p_ref, r_ref):
    g = pl.program_id(0)
    NG, B, KB = p_ref.shape
    GE, _, D_FF = w_ref.shape

    @pl.when(g == 0)
    def _build_dispatch():
        idx = idx_ref[...]
        ew = ew_ref[...]
        act = act_ref[...]
        a0t = jnp.tile(act[:, :D_FF], (1, GE))
        a1t = jnp.tile(act[:, D_FF:], (1, GE))
        v0 = ew[:, 0:1] * a0t
        v1 = ew[:, 1:2] * a1t
        colk = lax.broadcasted_iota(jnp.int32, (B, KB), 1) // D_FF
        for gg in range(NG):
            ce = colk + gg * GE
            pgg = (jnp.where(ce == idx[:, 0:1], v0, 0.0)
                   + jnp.where(ce == idx[:, 1:2], v1, 0.0))
            p_ref[gg] = pgg.astype(jnp.bfloat16)
        E = r_ref.shape[1]
        iota_e = lax.broadcasted_iota(jnp.int32, (B, E), 1)
        g0 = jnp.where(iota_e == idx[:, 0:1], ew[:, 0:1], 0.0)
        g1 = jnp.where(iota_e == idx[:, 1:2], ew[:, 1:2], 0.0)
        r_ref[...] = (g0 + g1).astype(jnp.bfloat16)

    wcat = jnp.concatenate(
        [w_ref[s] for s in range(GE)], axis=1).astype(jnp.bfloat16)
    contrib = lax.dot_general(
        p_ref[g], wcat, (((1,), (1,)), ((), ())),
        preferred_element_type=jnp.float32,
    )

    @pl.when(g == 0)
    def _init():
        bias_c = lax.dot_general(
            r_ref[...], bias_ref[...].astype(jnp.bfloat16),
            (((1,), (0,)), ((), ())), preferred_element_type=jnp.float32)
        out_ref[...] = resid_ref[...] + bias_c + contrib

    @pl.when(g != 0)
    def _acc():
        out_ref[...] += contrib


def kernel(activated, expert_indices, expert_weights, mlp2_weight, mlp2_bias, residual_x):
    B, TOPK, D_FF = activated.shape
    E, D_MODEL, _ = mlp2_weight.shape
    idx = jnp.asarray(expert_indices, jnp.int32)
    act2d = activated.reshape(B, TOPK * D_FF)

    GE = 64
    NG = E // GE
    return pl.pallas_call(
        _moe_fused,
        grid=(NG,),
        in_specs=[
            pl.BlockSpec((B, TOPK), lambda g: (0, 0)),
            pl.BlockSpec((B, TOPK), lambda g: (0, 0)),
            pl.BlockSpec((B, TOPK * D_FF), lambda g: (0, 0)),
            pl.BlockSpec((GE, D_MODEL, D_FF), lambda g: (g, 0, 0)),
            pl.BlockSpec((E, D_MODEL), lambda g: (0, 0)),
            pl.BlockSpec((B, D_MODEL), lambda g: (0, 0)),
        ],
        out_specs=pl.BlockSpec((B, D_MODEL), lambda g: (0, 0)),
        out_shape=jax.ShapeDtypeStruct((B, D_MODEL), jnp.float32),
        scratch_shapes=[
            pltpu.VMEM((NG, B, GE * D_FF), jnp.bfloat16),
            pltpu.VMEM((B, E), jnp.bfloat16),
        ],
    )(idx, expert_weights, act2d, mlp2_weight, mlp2_bias, residual_x)
